# baseline (speedup 1.0000x reference)
import numpy as np

import jax
import jax.numpy as jnp
from jax import lax
from jax.experimental import pallas as pl

S, DIM = 2048, 2048
NH, HD, RD = 16, 128, 64
QLR, TOPK = 1536, 1024
SCALE = float(HD) ** -0.5
WSCALE = float(NH) ** -0.5
QT = 256
KC = 256


def _hadamard_f32(n):
    h = np.array([[1.0]], dtype=np.float32)
    while h.shape[0] < n:
        h = np.block([[h, h], [h, -h]])
    return h

_HAD = jnp.asarray(_hadamard_f32(HD), dtype=jnp.float32)


def _kacc_kernel(x_ref, w_ref, out_ref):
    kk = pl.program_id(1)
    p = lax.dot_general(x_ref[...], w_ref[...], (((1,), (1,)), ((), ())),
                        preferred_element_type=jnp.float32)

    @pl.when(kk == 0)
    def _():
        out_ref[...] = p

    @pl.when(kk > 0)
    def _():
        out_ref[...] = out_ref[...] + p


def _kpost_kernel(kn_ref, fc_ref, fs_ref, had_ref, k_out):
    kn = kn_ref[...]
    fc = fc_ref[...]
    fs = fs_ref[...]
    k1 = kn[:, : RD // 2]
    k2 = kn[:, RD // 2 : RD]
    k_rot = jnp.concatenate([k1 * fc - k2 * fs, k1 * fs + k2 * fc, kn[:, RD:]], axis=1)
    k_out[...] = lax.dot_general(k_rot, had_ref[...], (((1,), (0,)), ((), ())),
                                 preferred_element_type=jnp.float32) * SCALE


def _qprep_kernel(qr_ref, wq_ref, w_ref, fc_ref, fs_ref, had_ref, qw_out):
    fc = fc_ref[...]
    fs = fs_ref[...]
    had = had_ref[...]
    q = lax.dot_general(qr_ref[...], wq_ref[...], (((1,), (1,)), ((), ())),
                        preferred_element_type=jnp.float32)
    wv = w_ref[...] * WSCALE
    for h in range(NH):
        qh = q[:, h * HD : (h + 1) * HD]
        q1 = qh[:, : RD // 2]
        q2 = qh[:, RD // 2 : RD]
        q_rot = jnp.concatenate([q1 * fc - q2 * fs, q1 * fs + q2 * fc, qh[:, RD:]], axis=1)
        qs = lax.dot_general(q_rot, had, (((1,), (0,)), ((), ())),
                             preferred_element_type=jnp.float32) * SCALE
        qw_out[:, h * HD : (h + 1) * HD] = (wv[:, h : h + 1] * qs) * SCALE


def _scores_kernel(qw_ref, k_ref, score_out):
    i = pl.program_id(0)
    kv = k_ref[...]
    hs = [lax.dot_general(qw_ref[:, h * HD : (h + 1) * HD], kv, (((1,), (1,)), ((), ())),
                          preferred_element_type=jnp.float32) for h in range(NH)]
    hs = [jnp.maximum(s, 0.0) for s in hs]
    while len(hs) > 1:
        hs = [hs[j] + hs[j + 1] for j in range(0, len(hs), 2)]
    acc = hs[0]
    row = i * QT + lax.broadcasted_iota(jnp.int32, (QT, S), 0)
    col = lax.broadcasted_iota(jnp.int32, (QT, S), 1)
    score_out[...] = jnp.where(col <= row, acc, acc + jnp.float32(-1e9))


def _cmp_stage(v, idx, j, desc_block, width, rows):
    col = lax.broadcasted_iota(jnp.int32, (rows, width), 1)
    low = (col & j) == 0
    pv = jnp.where(low, jnp.roll(v, -j, axis=1), jnp.roll(v, j, axis=1))
    pi = jnp.where(low, jnp.roll(idx, -j, axis=1), jnp.roll(idx, j, axis=1))
    mine_wins = (v > pv) | ((v == pv) & (idx < pi))
    take_mine = mine_wins == (desc_block == low)
    return jnp.where(take_mine, v, pv), jnp.where(take_mine, idx, pi)


def _topk_kernel(score_ref, idx_out):
    v = score_ref[...]
    idx = lax.broadcasted_iota(jnp.int32, (QT, S), 1)
    for k in [2 ** p for p in range(1, 11)]:
        j = k >> 1
        while j >= 1:
            col = lax.broadcasted_iota(jnp.int32, (QT, S), 1)
            desc_block = (col & k) == 0
            v, idx = _cmp_stage(v, idx, j, desc_block, S, QT)
            j >>= 1
    full_true = jnp.full((QT, S), True)
    v, idx = _cmp_stage(v, idx, TOPK, full_true, S, QT)
    v = v[:, :TOPK]
    idx = idx[:, :TOPK]
    half_true = jnp.full((QT, TOPK), True)
    j = TOPK >> 1
    while j >= 1:
        v, idx = _cmp_stage(v, idx, j, half_true, TOPK, QT)
        j >>= 1
    idx_out[...] = idx


def kernel(x, qr, start_pos, freqs_cos, freqs_sin, mask, wq_b_w, wk_w,
           k_norm_w, k_norm_b, weights_proj_w):
    del start_pos, mask
    x2 = x.reshape(S, DIM)
    qr2 = qr.reshape(S, QLR)
    grid = S // QT

    k_lin = pl.pallas_call(
        _kacc_kernel,
        grid=(grid, DIM // KC),
        in_specs=[
            pl.BlockSpec((QT, KC), lambda i, kk: (i, kk)),
            pl.BlockSpec((HD, KC), lambda i, kk: (0, kk)),
        ],
        out_specs=pl.BlockSpec((QT, HD), lambda i, kk: (i, 0)),
        out_shape=jax.ShapeDtypeStruct((S, HD), jnp.float32),
    )(x2, wk_w)

    w16 = pl.pallas_call(
        _kacc_kernel,
        grid=(grid, DIM // KC),
        in_specs=[
            pl.BlockSpec((QT, KC), lambda i, kk: (i, kk)),
            pl.BlockSpec((NH, KC), lambda i, kk: (0, kk)),
        ],
        out_specs=pl.BlockSpec((QT, NH), lambda i, kk: (i, 0)),
        out_shape=jax.ShapeDtypeStruct((S, NH), jnp.float32),
    )(x2, weights_proj_w)

    kl3 = k_lin.reshape(1, S, HD)
    mu = jnp.mean(kl3, axis=-1, keepdims=True)
    var = jnp.mean((kl3 - mu) ** 2, axis=-1, keepdims=True)
    kn = ((kl3 - mu) / jnp.sqrt(var + 1e-6) * k_norm_w + k_norm_b).reshape(S, HD)

    k = pl.pallas_call(
        _kpost_kernel,
        grid=(grid,),
        in_specs=[
            pl.BlockSpec((QT, HD), lambda i: (i, 0)),
            pl.BlockSpec((QT, RD // 2), lambda i: (i, 0)),
            pl.BlockSpec((QT, RD // 2), lambda i: (i, 0)),
            pl.BlockSpec((HD, HD), lambda i: (0, 0)),
        ],
        out_specs=pl.BlockSpec((QT, HD), lambda i: (i, 0)),
        out_shape=jax.ShapeDtypeStruct((S, HD), jnp.float32),
    )(kn, freqs_cos, freqs_sin, _HAD)

    qw = pl.pallas_call(
        _qprep_kernel,
        grid=(grid,),
        in_specs=[
            pl.BlockSpec((QT, QLR), lambda i: (i, 0)),
            pl.BlockSpec((NH * HD, QLR), lambda i: (0, 0)),
            pl.BlockSpec((QT, NH), lambda i: (i, 0)),
            pl.BlockSpec((QT, RD // 2), lambda i: (i, 0)),
            pl.BlockSpec((QT, RD // 2), lambda i: (i, 0)),
            pl.BlockSpec((HD, HD), lambda i: (0, 0)),
        ],
        out_specs=pl.BlockSpec((QT, NH * HD), lambda i: (i, 0)),
        out_shape=jax.ShapeDtypeStruct((S, NH * HD), jnp.float32),
    )(qr2, wq_b_w, w16, freqs_cos, freqs_sin, _HAD)

    score = pl.pallas_call(
        _scores_kernel,
        grid=(grid,),
        in_specs=[
            pl.BlockSpec((QT, NH * HD), lambda i: (i, 0)),
            pl.BlockSpec((S, HD), lambda i: (0, 0)),
        ],
        out_specs=pl.BlockSpec((QT, S), lambda i: (i, 0)),
        out_shape=jax.ShapeDtypeStruct((S, S), jnp.float32),
    )(qw, k)

    idx = pl.pallas_call(
        _topk_kernel,
        grid=(grid,),
        in_specs=[pl.BlockSpec((QT, S), lambda i: (i, 0))],
        out_specs=pl.BlockSpec((QT, TOPK), lambda i: (i, 0)),
        out_shape=jax.ShapeDtypeStruct((S, TOPK), jnp.int32),
    )(score)

    return idx[None]
